# Initial kernel scaffold; baseline (speedup 1.0000x reference)
import jax
import jax.numpy as jnp
from jax.experimental import pallas as pl


def kernel(inputs, token_table, position_table):
    raise NotImplementedError("write your pallas kernel here")


# speedup vs baseline: 1.3940x; 1.3940x over previous
import functools

import jax
import jax.numpy as jnp
from jax import lax
from jax.experimental import pallas as pl
from jax.experimental.pallas import tpu as pltpu
from jax.experimental.pallas import tpu_sc as plsc

VOCAB = 1000000
SEQ = 200
DIM = 32
BATCH = 4096
LANES = 16

NC = 2
NS = 16
NW = NC * NS

SEQ_PER_W = BATCH // NW
SEQ_PER_CHUNK = 4
CHUNK = SEQ_PER_CHUNK * SEQ
N_CHUNKS = SEQ_PER_W // SEQ_PER_CHUNK
ROWS_PER_W = SEQ_PER_W * SEQ


def _sc_body(tok_hbm, idx_hbm, pos_hbm, out_hbm, idx_v, rows_v, pos_v, gsem):
    wid = lax.axis_index("s") * NC + lax.axis_index("c")
    base = wid * ROWS_PER_W

    pltpu.sync_copy(pos_hbm, pos_v)

    def chunk_body(c, carry):
        cb = base + c * CHUNK
        pltpu.sync_copy(idx_hbm.at[pl.ds(cb, CHUNK)], idx_v)
        pltpu.async_copy(tok_hbm.at[idx_v], rows_v, gsem).wait()

        def row_body(v, carry2):
            p0 = pos_v[v, pl.ds(0, LANES)]
            p1 = pos_v[v, pl.ds(LANES, LANES)]
            for s in range(SEQ_PER_CHUNK):
                r = s * SEQ + v
                rows_v[r, pl.ds(0, LANES)] += p0
                rows_v[r, pl.ds(LANES, LANES)] += p1
            return carry2

        lax.fori_loop(0, SEQ, row_body, 0)
        pltpu.sync_copy(rows_v, out_hbm.at[pl.ds(cb, CHUNK)])
        return carry

    lax.fori_loop(0, N_CHUNKS, chunk_body, 0)


@jax.jit
def kernel(inputs, token_table, position_table):
    idx_flat = inputs.reshape(BATCH * SEQ).astype(jnp.int32)
    mesh = plsc.VectorSubcoreMesh(core_axis_name="c", subcore_axis_name="s")
    out = pl.kernel(
        _sc_body,
        mesh=mesh,
        compiler_params=pltpu.CompilerParams(use_tc_tiling_on_sc=False),
        out_type=jax.ShapeDtypeStruct((BATCH * SEQ, DIM), jnp.float32),
        scratch_types=[
            pltpu.VMEM((CHUNK,), jnp.int32),
            pltpu.VMEM((CHUNK, DIM), jnp.float32),
            pltpu.VMEM((SEQ, DIM), jnp.float32),
            pltpu.SemaphoreType.DMA,
        ],
    )(token_table, idx_flat, position_table)
    return out.reshape(BATCH, SEQ, DIM)
